# baseline (speedup 1.0000x reference)
import math

import jax
import jax.numpy as jnp
from jax.experimental import pallas as pl
from jax.experimental.pallas import tpu as pltpu

NUM_HEADS = 16
NUM_BUCKETS = 32
MAX_DISTANCE = 128
Q = 2048
K = 2048
BR = 64
WPAD = 4224


def _expand_kernel(tab_ref, out_ref, tpad_ref, tt_ref):
    @pl.when(pl.program_id(0) == 0)
    def _precompute():
        delta = jax.lax.broadcasted_iota(jnp.int32, (1, WPAD), 1) - (Q - 1)
        half = NUM_BUCKETS // 2
        rel_buckets = (delta > 0).astype(jnp.int32) * half
        a = jnp.abs(delta)
        max_exact = half // 2
        is_small = a < max_exact
        rel_large = max_exact + (
            jnp.log(a.astype(jnp.float32) / max_exact)
            / math.log(MAX_DISTANCE / max_exact)
            * (half - max_exact)
        ).astype(jnp.int32)
        rel_large = jnp.minimum(rel_large, half - 1)
        bucket = rel_buckets + jnp.where(is_small, a, rel_large)
        acc = jnp.zeros((NUM_HEADS, WPAD), dtype=jnp.float32)
        for b in range(NUM_BUCKETS):
            acc = jnp.where(bucket == b, tab_ref[:, b : b + 1], acc)
        tpad_ref[...] = acc
        for m in range(16):
            for k in range(8):
                s = 8 * m + 7 - k
                tt_ref[:, m, k, :] = tpad_ref[:, s : s + 4096]

    i0 = pl.program_id(0) * BR
    for g in range(BR // 8):
        b = (Q - 8) - (i0 + 8 * g)
        m = (b // 8) % 16
        off = pl.multiple_of((b // 128) * 128, 128)
        out_ref[0, :, 8 * g : 8 * g + 8, :] = tt_ref[:, m, :, pl.ds(off, K)]


def kernel(bias_table, query_length, key_length):
    del query_length, key_length
    tab_t = bias_table.T
    return pl.pallas_call(
        _expand_kernel,
        grid=(Q // BR,),
        in_specs=[pl.BlockSpec((NUM_HEADS, NUM_BUCKETS), lambda i: (0, 0))],
        out_specs=pl.BlockSpec((1, NUM_HEADS, BR, K), lambda i: (0, 0, i, 0)),
        out_shape=jax.ShapeDtypeStruct((1, NUM_HEADS, Q, K), jnp.float32),
        scratch_shapes=[
            pltpu.VMEM((NUM_HEADS, WPAD), jnp.float32),
            pltpu.VMEM((NUM_HEADS, 16, 8, 4096), jnp.float32),
        ],
    )(tab_t)


# speedup vs baseline: 1.1187x; 1.1187x over previous
import math

import jax
import jax.numpy as jnp
from jax.experimental import pallas as pl
from jax.experimental.pallas import tpu as pltpu

NUM_HEADS = 16
NUM_BUCKETS = 32
MAX_DISTANCE = 128
Q = 2048
K = 2048
WPAD = 4224
LAG = 32


def _expand_kernel(tab_ref, out_ref, tpad_ref, tt_ref, sem):
    delta = jax.lax.broadcasted_iota(jnp.int32, (1, WPAD), 1) - (Q - 1)
    half = NUM_BUCKETS // 2
    rel_buckets = (delta > 0).astype(jnp.int32) * half
    a = jnp.abs(delta)
    max_exact = half // 2
    is_small = a < max_exact
    rel_large = max_exact + (
        jnp.log(a.astype(jnp.float32) / max_exact)
        / math.log(MAX_DISTANCE / max_exact)
        * (half - max_exact)
    ).astype(jnp.int32)
    rel_large = jnp.minimum(rel_large, half - 1)
    bucket = rel_buckets + jnp.where(is_small, a, rel_large)
    acc = jnp.zeros((NUM_HEADS, WPAD), dtype=jnp.float32)
    for b in range(NUM_BUCKETS):
        acc = jnp.where(bucket == b, tab_ref[:, b : b + 1], acc)
    tpad_ref[...] = acc

    pending = []
    for m in range(16):
        for k in range(8):
            s = 8 * m + 7 - k
            tt_ref[:, m, k, :] = tpad_ref[:, s : s + 4096]
        for c in range(16):
            i = (Q - 8) - (128 * c + 8 * m)
            cp = pltpu.make_async_copy(
                tt_ref.at[:, m, :, pl.ds(128 * c, K)],
                out_ref.at[0, :, pl.ds(i, 8), :],
                sem,
            )
            cp.start()
            pending.append(cp)
            if len(pending) > LAG:
                pending.pop(0).wait()
    for cp in pending:
        cp.wait()


def kernel(bias_table, query_length, key_length):
    del query_length, key_length
    tab_t = bias_table.T
    return pl.pallas_call(
        _expand_kernel,
        in_specs=[pl.BlockSpec(memory_space=pltpu.VMEM)],
        out_specs=pl.BlockSpec(memory_space=pl.ANY),
        out_shape=jax.ShapeDtypeStruct((1, NUM_HEADS, Q, K), jnp.float32),
        scratch_shapes=[
            pltpu.VMEM((NUM_HEADS, WPAD), jnp.float32),
            pltpu.VMEM((NUM_HEADS, 16, 8, 4096), jnp.float32),
            pltpu.SemaphoreType.DMA,
        ],
    )(tab_t)


# speedup vs baseline: 1.1198x; 1.0010x over previous
import math

import jax
import jax.numpy as jnp
from jax.experimental import pallas as pl
from jax.experimental.pallas import tpu as pltpu

NUM_HEADS = 16
NUM_BUCKETS = 32
MAX_DISTANCE = 128
Q = 2048
K = 2048
WPAD = 4224
LAG = 128


def _expand_kernel(tab_ref, out_ref, tpad_ref, tt_ref, sem):
    delta = jax.lax.broadcasted_iota(jnp.int32, (1, WPAD), 1) - (Q - 1)
    half = NUM_BUCKETS // 2
    rel_buckets = (delta > 0).astype(jnp.int32) * half
    a = jnp.abs(delta)
    max_exact = half // 2
    is_small = a < max_exact
    rel_large = max_exact + (
        jnp.log(a.astype(jnp.float32) / max_exact)
        / math.log(MAX_DISTANCE / max_exact)
        * (half - max_exact)
    ).astype(jnp.int32)
    rel_large = jnp.minimum(rel_large, half - 1)
    bucket = rel_buckets + jnp.where(is_small, a, rel_large)
    acc = jnp.zeros((NUM_HEADS, WPAD), dtype=jnp.float32)
    for b in range(NUM_BUCKETS):
        acc = jnp.where(bucket == b, tab_ref[:, b : b + 1], acc)
    tpad_ref[...] = acc

    pending = []
    for m in range(16):
        for k in range(8):
            s = 8 * m + 7 - k
            tt_ref[:, m, k, :] = tpad_ref[:, s : s + 4096]
        for c in range(16):
            i = (Q - 8) - (128 * c + 8 * m)
            cp = pltpu.make_async_copy(
                tt_ref.at[:, m, :, pl.ds(128 * c, K)],
                out_ref.at[0, :, pl.ds(i, 8), :],
                sem,
            )
            cp.start()
            pending.append(cp)
            if len(pending) > LAG:
                pending.pop(0).wait()
    for cp in pending:
        cp.wait()


def kernel(bias_table, query_length, key_length):
    del query_length, key_length
    tab_t = bias_table.T
    return pl.pallas_call(
        _expand_kernel,
        in_specs=[pl.BlockSpec(memory_space=pltpu.VMEM)],
        out_specs=pl.BlockSpec(memory_space=pl.ANY),
        out_shape=jax.ShapeDtypeStruct((1, NUM_HEADS, Q, K), jnp.float32),
        scratch_shapes=[
            pltpu.VMEM((NUM_HEADS, WPAD), jnp.float32),
            pltpu.VMEM((NUM_HEADS, 16, 8, 4096), jnp.float32),
            pltpu.SemaphoreType.DMA,
        ],
    )(tab_t)
